# Initial kernel scaffold; baseline (speedup 1.0000x reference)
import jax
import jax.numpy as jnp
from jax.experimental import pallas as pl


def kernel(x, edge_attr, graph_attr, We, be, Wn, bn, edge_index, batch):
    raise NotImplementedError("write your pallas kernel here")


# speedup vs baseline: 2.1563x; 2.1563x over previous
import functools

import jax
import jax.numpy as jnp
from jax.experimental import pallas as pl

N = 50000
E = 800000
D = 64
G = 16

_BN = 1000
_BE = 8000


def _precomp_body(x_ref, batch_ref, wes_ref, wed_ref, weg_ref, be_ref, ga_ref,
                  xs_ref, xdg_ref):
    xb = x_ref[...]
    xs_ref[...] = jnp.dot(xb, wes_ref[...], preferred_element_type=jnp.float32)
    gg = jnp.dot(ga_ref[...], weg_ref[...], preferred_element_type=jnp.float32)
    oh = (batch_ref[...] == jax.lax.broadcasted_iota(jnp.int32, (1, G), 1)
          ).astype(jnp.float32)
    xdg_ref[...] = (jnp.dot(xb, wed_ref[...], preferred_element_type=jnp.float32)
                    + jnp.dot(oh, gg, preferred_element_type=jnp.float32)
                    + be_ref[...])


def _edge_mm_body(ea_ref, wee_ref, pe_ref):
    pe_ref[...] = jnp.dot(ea_ref[...], wee_ref[...],
                          preferred_element_type=jnp.float32)


def _node_body(x_ref, agg_ref, batch_ref, wnx_ref, wna_ref, wng_ref, bn_ref,
               ga_ref, ne_ref, ge_ref):
    i = pl.program_id(0)
    xb = x_ref[...]
    ab = agg_ref[...]
    gb = jnp.dot(ga_ref[...], wng_ref[...], preferred_element_type=jnp.float32)
    oh = (batch_ref[...] == jax.lax.broadcasted_iota(jnp.int32, (1, G), 1)
          ).astype(jnp.float32)
    ne = jnp.maximum(
        jnp.dot(xb, wnx_ref[...], preferred_element_type=jnp.float32)
        + jnp.dot(ab, wna_ref[...], preferred_element_type=jnp.float32)
        + jnp.dot(oh, gb, preferred_element_type=jnp.float32)
        + bn_ref[...], 0.0)
    ne_ref[...] = ne
    masked = jnp.where(oh[:, :, None] > 0, ne[:, None, :], 0.0)
    part = jnp.max(masked, axis=0)

    @pl.when(i == 0)
    def _init():
        ge_ref[...] = part

    @pl.when(i > 0)
    def _acc():
        ge_ref[...] = jnp.maximum(ge_ref[...], part)


def _full(shape):
    return pl.BlockSpec(shape, lambda i: (0,) * len(shape))


def kernel(x, edge_attr, graph_attr, We, be, Wn, bn, edge_index, batch):
    we_s, we_d, we_e, we_g = We[0:D], We[D:2 * D], We[2 * D:3 * D], We[3 * D:]
    wn_x, wn_a, wn_g = Wn[0:D], Wn[D:2 * D], Wn[2 * D:]
    be2 = be.reshape(1, D)
    bn2 = bn.reshape(1, D)
    batch2 = batch.reshape(N, 1)

    xs, xdg = pl.pallas_call(
        _precomp_body,
        grid=(N // _BN,),
        in_specs=[
            pl.BlockSpec((_BN, D), lambda i: (i, 0)),
            pl.BlockSpec((_BN, 1), lambda i: (i, 0)),
            _full((D, D)), _full((D, D)), _full((D, D)),
            _full((1, D)), _full((G, D)),
        ],
        out_specs=[
            pl.BlockSpec((_BN, D), lambda i: (i, 0)),
            pl.BlockSpec((_BN, D), lambda i: (i, 0)),
        ],
        out_shape=[
            jax.ShapeDtypeStruct((N, D), jnp.float32),
            jax.ShapeDtypeStruct((N, D), jnp.float32),
        ],
    )(x, batch2, we_s, we_d, we_g, be2, graph_attr)

    pe = pl.pallas_call(
        _edge_mm_body,
        grid=(E // _BE,),
        in_specs=[pl.BlockSpec((_BE, D), lambda i: (i, 0)), _full((D, D))],
        out_specs=pl.BlockSpec((_BE, D), lambda i: (i, 0)),
        out_shape=jax.ShapeDtypeStruct((E, D), jnp.float32),
    )(edge_attr, we_e)

    src = edge_index[0]
    dst = edge_index[1]
    ee = jnp.maximum(jnp.take(xs, src, axis=0) + jnp.take(xdg, dst, axis=0)
                     + pe, 0.0)
    agg = jax.ops.segment_max(ee, dst, num_segments=N)
    agg = jnp.where(jnp.isfinite(agg), agg, 0.0)

    ne, ge = pl.pallas_call(
        _node_body,
        grid=(N // _BN,),
        in_specs=[
            pl.BlockSpec((_BN, D), lambda i: (i, 0)),
            pl.BlockSpec((_BN, D), lambda i: (i, 0)),
            pl.BlockSpec((_BN, 1), lambda i: (i, 0)),
            _full((D, D)), _full((D, D)), _full((D, D)),
            _full((1, D)), _full((G, D)),
        ],
        out_specs=[
            pl.BlockSpec((_BN, D), lambda i: (i, 0)),
            pl.BlockSpec((G, D), lambda i: (0, 0)),
        ],
        out_shape=[
            jax.ShapeDtypeStruct((N, D), jnp.float32),
            jax.ShapeDtypeStruct((G, D), jnp.float32),
        ],
    )(x, agg, batch2, wn_x, wn_a, wn_g, bn2, graph_attr)

    return (ne, ee, ge)


# speedup vs baseline: 3.6478x; 1.6917x over previous
import functools

import jax
import jax.numpy as jnp
from jax import lax
from jax.experimental import pallas as pl
from jax.experimental.pallas import tpu as pltpu
from jax.experimental.pallas import tpu_sc as plsc

N = 50000
E = 800000
D = 64
G = 16

_BN = 1000
_BE = 8000

_NW = 32
_CHUNK = E // _NW
_B = 128
_NFULL = _CHUNK // _B
_REM = _CHUNK - _NFULL * _B


def _edge_sc_body(xs_hbm, xdg_hbm, pe_hbm, src_hbm, dst_hbm, ee_hbm,
                  srcv, dstv, srcr, dstr, xsr, xdr, pev, xsr2, xdr2, pev2,
                  sem):
    wid = lax.axis_index("s") * 2 + lax.axis_index("c")
    cbase = wid * _CHUNK

    def process(base, nb, sv, dv, xs_b, xd_b, pe_b):
        pltpu.sync_copy(src_hbm.at[pl.ds(base, nb)], sv)
        pltpu.sync_copy(dst_hbm.at[pl.ds(base, nb)], dv)
        pltpu.async_copy(xs_hbm.at[sv], xs_b, sem).wait()
        pltpu.async_copy(xdg_hbm.at[dv], xd_b, sem).wait()
        pltpu.sync_copy(pe_hbm.at[pl.ds(base, nb)], pe_b)

        def row(r, _):
            for c in range(4):
                s = pl.ds(c * 16, 16)
                pe_b[r, s] = jnp.maximum(xs_b[r, s] + xd_b[r, s] + pe_b[r, s],
                                         0.0)
            return 0

        lax.fori_loop(0, nb, row, 0)
        pltpu.sync_copy(pe_b, ee_hbm.at[pl.ds(base, nb)])

    def blk(j, _):
        process(cbase + j * _B, _B, srcv, dstv, xsr, xdr, pev)
        return 0

    lax.fori_loop(0, _NFULL, blk, 0)
    process(cbase + _NFULL * _B, _REM, srcr, dstr, xsr2, xdr2, pev2)


def _edge_sc(xs, xdg, pe, src, dst):
    mesh = plsc.VectorSubcoreMesh(core_axis_name="c", subcore_axis_name="s")
    return pl.kernel(
        _edge_sc_body,
        mesh=mesh,
        compiler_params=pltpu.CompilerParams(use_tc_tiling_on_sc=False),
        out_type=jax.ShapeDtypeStruct((E, D), jnp.float32),
        scratch_types=[
            pltpu.VMEM((_B,), jnp.int32),
            pltpu.VMEM((_B,), jnp.int32),
            pltpu.VMEM((_REM,), jnp.int32),
            pltpu.VMEM((_REM,), jnp.int32),
            pltpu.VMEM((_B, D), jnp.float32),
            pltpu.VMEM((_B, D), jnp.float32),
            pltpu.VMEM((_B, D), jnp.float32),
            pltpu.VMEM((_REM, D), jnp.float32),
            pltpu.VMEM((_REM, D), jnp.float32),
            pltpu.VMEM((_REM, D), jnp.float32),
            pltpu.SemaphoreType.DMA,
        ],
    )(xs, xdg, pe, src, dst)


def _precomp_body(x_ref, batch_ref, wes_ref, wed_ref, weg_ref, be_ref, ga_ref,
                  xs_ref, xdg_ref):
    xb = x_ref[...]
    xs_ref[...] = jnp.dot(xb, wes_ref[...], preferred_element_type=jnp.float32)
    gg = jnp.dot(ga_ref[...], weg_ref[...], preferred_element_type=jnp.float32)
    oh = (batch_ref[...] == jax.lax.broadcasted_iota(jnp.int32, (1, G), 1)
          ).astype(jnp.float32)
    xdg_ref[...] = (jnp.dot(xb, wed_ref[...], preferred_element_type=jnp.float32)
                    + jnp.dot(oh, gg, preferred_element_type=jnp.float32)
                    + be_ref[...])


def _edge_mm_body(ea_ref, wee_ref, pe_ref):
    pe_ref[...] = jnp.dot(ea_ref[...], wee_ref[...],
                          preferred_element_type=jnp.float32)


def _node_body(x_ref, agg_ref, batch_ref, wnx_ref, wna_ref, wng_ref, bn_ref,
               ga_ref, ne_ref, ge_ref):
    i = pl.program_id(0)
    xb = x_ref[...]
    ab = agg_ref[...]
    gb = jnp.dot(ga_ref[...], wng_ref[...], preferred_element_type=jnp.float32)
    oh = (batch_ref[...] == jax.lax.broadcasted_iota(jnp.int32, (1, G), 1)
          ).astype(jnp.float32)
    ne = jnp.maximum(
        jnp.dot(xb, wnx_ref[...], preferred_element_type=jnp.float32)
        + jnp.dot(ab, wna_ref[...], preferred_element_type=jnp.float32)
        + jnp.dot(oh, gb, preferred_element_type=jnp.float32)
        + bn_ref[...], 0.0)
    ne_ref[...] = ne
    masked = jnp.where(oh[:, :, None] > 0, ne[:, None, :], 0.0)
    part = jnp.max(masked, axis=0)

    @pl.when(i == 0)
    def _init():
        ge_ref[...] = part

    @pl.when(i > 0)
    def _acc():
        ge_ref[...] = jnp.maximum(ge_ref[...], part)


def _full(shape):
    return pl.BlockSpec(shape, lambda i: (0,) * len(shape))


def kernel(x, edge_attr, graph_attr, We, be, Wn, bn, edge_index, batch):
    we_s, we_d, we_e, we_g = We[0:D], We[D:2 * D], We[2 * D:3 * D], We[3 * D:]
    wn_x, wn_a, wn_g = Wn[0:D], Wn[D:2 * D], Wn[2 * D:]
    be2 = be.reshape(1, D)
    bn2 = bn.reshape(1, D)
    batch2 = batch.reshape(N, 1)

    xs, xdg = pl.pallas_call(
        _precomp_body,
        grid=(N // _BN,),
        in_specs=[
            pl.BlockSpec((_BN, D), lambda i: (i, 0)),
            pl.BlockSpec((_BN, 1), lambda i: (i, 0)),
            _full((D, D)), _full((D, D)), _full((D, D)),
            _full((1, D)), _full((G, D)),
        ],
        out_specs=[
            pl.BlockSpec((_BN, D), lambda i: (i, 0)),
            pl.BlockSpec((_BN, D), lambda i: (i, 0)),
        ],
        out_shape=[
            jax.ShapeDtypeStruct((N, D), jnp.float32),
            jax.ShapeDtypeStruct((N, D), jnp.float32),
        ],
    )(x, batch2, we_s, we_d, we_g, be2, graph_attr)

    pe = pl.pallas_call(
        _edge_mm_body,
        grid=(E // _BE,),
        in_specs=[pl.BlockSpec((_BE, D), lambda i: (i, 0)), _full((D, D))],
        out_specs=pl.BlockSpec((_BE, D), lambda i: (i, 0)),
        out_shape=jax.ShapeDtypeStruct((E, D), jnp.float32),
    )(edge_attr, we_e)

    src = edge_index[0]
    dst = edge_index[1]
    ee = _edge_sc(xs, xdg, pe, src, dst)
    agg = jax.ops.segment_max(ee, dst, num_segments=N)
    agg = jnp.where(jnp.isfinite(agg), agg, 0.0)

    ne, ge = pl.pallas_call(
        _node_body,
        grid=(N // _BN,),
        in_specs=[
            pl.BlockSpec((_BN, D), lambda i: (i, 0)),
            pl.BlockSpec((_BN, D), lambda i: (i, 0)),
            pl.BlockSpec((_BN, 1), lambda i: (i, 0)),
            _full((D, D)), _full((D, D)), _full((D, D)),
            _full((1, D)), _full((G, D)),
        ],
        out_specs=[
            pl.BlockSpec((_BN, D), lambda i: (i, 0)),
            pl.BlockSpec((G, D), lambda i: (0, 0)),
        ],
        out_shape=[
            jax.ShapeDtypeStruct((N, D), jnp.float32),
            jax.ShapeDtypeStruct((G, D), jnp.float32),
        ],
    )(x, agg, batch2, wn_x, wn_a, wn_g, bn2, graph_attr)

    return (ne, ee, ge)
